# baseline (speedup 1.0000x reference)
import jax
import jax.numpy as jnp
from jax import lax
from jax.experimental import pallas as pl
from jax.experimental.pallas import tpu as pltpu
from jax.experimental.pallas import tpu_sc as plsc

INPUT_DIM = 512
NUM_NEURONS = 512
BATCH = 1024
TIMESTEPS = 100

_NC = 2
_NS = 16
_NW = _NC * _NS
_ROWS = BATCH // _NW
_NSL = INPUT_DIM // 16
_PLANE = BATCH * NUM_NEURONS
_CHUNK = _ROWS * NUM_NEURONS


def _body(x_hbm, out_hbm, x_v, st_v, zbuf, ones_v, dummy_v, sem_z, sem_s):
    wid = lax.axis_index("s") * _NC + lax.axis_index("c")
    base = wid * _ROWS
    pltpu.sync_copy(x_hbm.at[pl.ds(base, _ROWS)], x_v)

    zero_f = jnp.zeros((16,), jnp.float32)
    lane = lax.iota(jnp.int32, 16)
    ones_v[...] = jnp.ones((16,), jnp.float32)

    def _zb(i, _):
        zbuf[pl.ds(i * 16, 16)] = zero_f
        return 0

    lax.fori_loop(0, _CHUNK // 16, _zb, 0)

    def _zfire(t, _):
        pltpu.make_async_copy(
            zbuf, out_hbm.at[pl.ds(t * _PLANE + base * NUM_NEURONS, _CHUNK)], sem_z
        ).start()
        return 0

    lax.fori_loop(0, TIMESTEPS, _zfire, 0)

    def _st(s, _):
        r = s // _NSL
        c = (s % _NSL) * 16
        xs = x_v[r, pl.ds(c, 16)]
        sig = 1.0 / (1.0 + jnp.exp(-xs))
        st_v[r, pl.ds(c, 16)] = (sig * jnp.float32(TIMESTEPS - 1)).astype(jnp.int32)
        return 0

    lax.fori_loop(0, _ROWS * _NSL, _st, 0)

    def _zdrain(t, _):
        pltpu.make_async_copy(zbuf, out_hbm.at[pl.ds(0, _CHUNK)], sem_z).wait()
        return 0

    lax.fori_loop(0, TIMESTEPS, _zdrain, 0)

    _LAG = 64

    def _issue(s):
        r = s // _NSL
        c = (s % _NSL) * 16
        st = st_v[r, pl.ds(c, 16)]
        idx = st * _PLANE + (base + r) * NUM_NEURONS + c + lane
        pltpu.make_async_copy(ones_v, out_hbm.at[idx], sem_s).start()

    def _sdrain1():
        pltpu.make_async_copy(x_hbm.at[0, pl.ds(0, 16)], dummy_v, sem_s).wait()

    def _s0(s, _):
        _issue(s)
        return 0

    lax.fori_loop(0, _LAG, _s0, 0)

    def _s1(s, _):
        _sdrain1()
        _issue(s)
        return 0

    lax.fori_loop(_LAG, _ROWS * _NSL, _s1, 0)

    def _s2(s, _):
        _sdrain1()
        return 0

    lax.fori_loop(0, _LAG, _s2, 0)


def kernel(continuous_input, timesteps):
    del timesteps
    mesh = plsc.VectorSubcoreMesh(core_axis_name="c", subcore_axis_name="s")
    run = pl.kernel(
        _body,
        out_type=jax.ShapeDtypeStruct((TIMESTEPS * BATCH * NUM_NEURONS,), jnp.float32),
        mesh=mesh,
        scratch_types=[
            pltpu.VMEM((_ROWS, INPUT_DIM), jnp.float32),
            pltpu.VMEM((_ROWS, INPUT_DIM), jnp.int32),
            pltpu.VMEM((_CHUNK,), jnp.float32),
            pltpu.VMEM((16,), jnp.float32),
            pltpu.VMEM((16,), jnp.float32),
            pltpu.SemaphoreType.DMA,
            pltpu.SemaphoreType.DMA,
        ],
        compiler_params=pltpu.CompilerParams(
            use_tc_tiling_on_sc=False, needs_layout_passes=False
        ),
    )
    flat = run(continuous_input)
    return flat.reshape(TIMESTEPS, BATCH, NUM_NEURONS)
